# baseline (speedup 1.0000x reference)
import jax
import jax.numpy as jnp
from jax import lax
from jax.experimental import pallas as pl
from jax.experimental.pallas import tpu as pltpu
from jax.experimental.pallas import tpu_sc as plsc

WEIGHT_ROWS = 4096
WEIGHT_COLS = 4096
CODE_DIM = 8
NUM_CODES = 8192
NUM_VECS = (WEIGHT_ROWS * WEIGHT_COLS) // CODE_DIM

NC, NS, L = 2, 16, 16
NW = NC * NS
CH = 8192
CODE_W = CODE_DIM // 2
ROW_W = WEIGHT_COLS // 2
CH_ROWS = CH * CODE_W // ROW_W

N_SPLIT = 2
VECS_PER_CALL = NUM_VECS // N_SPLIT
ROWS_PER_CALL = WEIGHT_ROWS // N_SPLIT


def _make_sc_gather_body(base_vec):
    vecs_per_w = VECS_PER_CALL // NW
    n_chunks = vecs_per_w // CH

    def body(cb_hbm, idx_hbm, out_hbm, cb_v, idx_v, rows_v):
        wid = lax.axis_index("s") * NC + lax.axis_index("c")
        base = base_vec + wid * vecs_per_w
        pltpu.sync_copy(cb_hbm, cb_v)
        lanes_w = lax.iota(jnp.int32, L) * CODE_W

        def chunk(g, carry):
            cbase = base + g * CH
            pltpu.sync_copy(idx_hbm.at[pl.ds(cbase, CH)], idx_v)

            def step(i, carry2):
                idx16 = idx_v[pl.ds(i * L, L)] * CODE_W
                row16 = jnp.full((L,), i // (ROW_W // (L * CODE_W)), jnp.int32)
                colbase = lanes_w + (i % (ROW_W // (L * CODE_W))) * (L * CODE_W)
                for d in range(CODE_W):
                    vals = plsc.load_gather(cb_v, [idx16 + d])
                    plsc.store_scatter(rows_v, [row16, colbase + d], vals)
                return carry2

            lax.fori_loop(0, CH // L, step, 0)
            row0 = pl.multiple_of((cbase - base_vec) // (CH // CH_ROWS), CH_ROWS)
            pltpu.sync_copy(rows_v, out_hbm.at[pl.ds(row0, CH_ROWS)])
            return carry

        lax.fori_loop(0, n_chunks, chunk, 0)

    return body


def _sc_gather(cb_words, indices, base_vec):
    mesh = plsc.VectorSubcoreMesh(core_axis_name="c", subcore_axis_name="s")
    fn = pl.kernel(
        _make_sc_gather_body(base_vec),
        out_type=jax.ShapeDtypeStruct((ROWS_PER_CALL, ROW_W), jnp.int32),
        mesh=mesh,
        scratch_types=[
            pltpu.VMEM((NUM_CODES * CODE_W,), jnp.int32),
            pltpu.VMEM((CH,), jnp.int32),
            pltpu.VMEM((CH_ROWS, ROW_W), jnp.int32),
        ],
        compiler_params=pltpu.CompilerParams(needs_layout_passes=False),
    )
    return fn(cb_words, indices)


BM, BK = 2048, 1024
BKW = BK // 2
BN = ROWS_PER_CALL
_HI_MASK = -65536


def _mm_compute(x_ref, g_ref, s_ref, o_ref):
    k = pl.program_id(1)

    @pl.when(k == 0)
    def _zero():
        o_ref[...] = jnp.zeros_like(o_ref)

    w = g_ref[...]
    g_even = pltpu.bitcast(w << 16, jnp.float32).astype(jnp.bfloat16)
    g_odd = pltpu.bitcast(w & _HI_MASK, jnp.float32).astype(jnp.bfloat16)
    g2 = jnp.concatenate([g_even, g_odd], axis=1)
    o_ref[...] += lax.dot_general(
        x_ref[...], g2, (((1,), (1,)), ((), ())),
        preferred_element_type=jnp.float32,
        precision=lax.Precision.DEFAULT,
    )

    @pl.when(k == pl.num_programs(1) - 1)
    def _scale():
        o_ref[...] *= s_ref[...]


def _mm_body_first(x_ref, g_ref, s_ref, o_ref):
    _mm_compute(x_ref, g_ref, s_ref, o_ref)


def _mm_body_rest(x_ref, g_ref, s_ref, prev_ref, o_ref):
    _mm_compute(x_ref, g_ref, s_ref, o_ref)


def _tc_matmul_part(x, g_words, s_row, prev, col):
    m, kdim = x.shape
    grid = (m // BM, kdim // BK)
    in_specs = [
        pl.BlockSpec((BM, BK), lambda i, k: (i, k)),
        pl.BlockSpec((BN, BKW), lambda i, k: (0, k)),
        pl.BlockSpec((1, BN), lambda i, k: (0, 0)),
    ]
    args = (x, g_words, s_row)
    if prev is None:
        body, aliases = _mm_body_first, {}
    else:
        body, aliases = _mm_body_rest, {3: 0}
        in_specs.append(pl.BlockSpec(memory_space=pl.ANY))
        args = args + (prev,)
    return pl.pallas_call(
        body,
        grid=grid,
        in_specs=in_specs,
        out_specs=pl.BlockSpec((BM, BN), lambda i, k, c=col: (i, c)),
        out_shape=jax.ShapeDtypeStruct((m, WEIGHT_ROWS), jnp.float32),
        input_output_aliases=aliases,
        compiler_params=pltpu.CompilerParams(
            dimension_semantics=("parallel", "arbitrary"),
            vmem_limit_bytes=100 * 1024 * 1024,
        ),
    )(*args)


def kernel(x, indices, codebook, scales):
    cb_words = lax.bitcast_convert_type(
        codebook.astype(jnp.bfloat16).reshape(NUM_CODES, CODE_W, 2), jnp.int32
    ).reshape(-1)
    perm = (
        jnp.arange(WEIGHT_COLS, dtype=jnp.int32)
        .reshape(WEIGHT_COLS // BK, BKW, 2)
        .transpose(0, 2, 1)
        .reshape(-1)
    )
    x_bf = jnp.take(x.astype(jnp.bfloat16), perm, axis=1)
    s_row = scales.reshape(1, WEIGHT_ROWS)
    out = None
    for part in range(N_SPLIT):
        gw = _sc_gather(cb_words, indices, part * VECS_PER_CALL)
        s_part = lax.slice_in_dim(s_row, part * BN, (part + 1) * BN, axis=1)
        out = _tc_matmul_part(x_bf, gw, s_part, out, part)
    return out


# speedup vs baseline: 1.0078x; 1.0078x over previous
import jax
import jax.numpy as jnp
from jax import lax
from jax.experimental import pallas as pl
from jax.experimental.pallas import tpu as pltpu
from jax.experimental.pallas import tpu_sc as plsc

WEIGHT_ROWS = 4096
WEIGHT_COLS = 4096
CODE_DIM = 8
NUM_CODES = 8192
NUM_VECS = (WEIGHT_ROWS * WEIGHT_COLS) // CODE_DIM

NC, NS, L = 2, 16, 16
NW = NC * NS
CH = 16384
CODE_W = CODE_DIM // 2
ROW_W = WEIGHT_COLS // 2
CH_ROWS = CH * CODE_W // ROW_W

N_SPLIT = 2
VECS_PER_CALL = NUM_VECS // N_SPLIT
ROWS_PER_CALL = WEIGHT_ROWS // N_SPLIT


def _make_sc_gather_body(base_vec):
    vecs_per_w = VECS_PER_CALL // NW
    n_chunks = vecs_per_w // CH

    def body(cb_hbm, idx_hbm, out_hbm, cb_v, idx_v, rows_v):
        wid = lax.axis_index("s") * NC + lax.axis_index("c")
        base = base_vec + wid * vecs_per_w
        pltpu.sync_copy(cb_hbm, cb_v)
        lanes_w = lax.iota(jnp.int32, L) * CODE_W

        def chunk(g, carry):
            cbase = base + g * CH
            pltpu.sync_copy(idx_hbm.at[pl.ds(cbase, CH)], idx_v)

            def step(i, carry2):
                idx16 = idx_v[pl.ds(i * L, L)] * CODE_W
                row16 = jnp.full((L,), i // (ROW_W // (L * CODE_W)), jnp.int32)
                colbase = lanes_w + (i % (ROW_W // (L * CODE_W))) * (L * CODE_W)
                for d in range(CODE_W):
                    vals = plsc.load_gather(cb_v, [idx16 + d])
                    plsc.store_scatter(rows_v, [row16, colbase + d], vals)
                return carry2

            lax.fori_loop(0, CH // L, step, 0)
            row0 = pl.multiple_of((cbase - base_vec) // (CH // CH_ROWS), CH_ROWS)
            pltpu.sync_copy(rows_v, out_hbm.at[pl.ds(row0, CH_ROWS)])
            return carry

        lax.fori_loop(0, n_chunks, chunk, 0)

    return body


def _sc_gather(cb_words, indices, base_vec):
    mesh = plsc.VectorSubcoreMesh(core_axis_name="c", subcore_axis_name="s")
    fn = pl.kernel(
        _make_sc_gather_body(base_vec),
        out_type=jax.ShapeDtypeStruct((ROWS_PER_CALL, ROW_W), jnp.int32),
        mesh=mesh,
        scratch_types=[
            pltpu.VMEM((NUM_CODES * CODE_W,), jnp.int32),
            pltpu.VMEM((CH,), jnp.int32),
            pltpu.VMEM((CH_ROWS, ROW_W), jnp.int32),
        ],
        compiler_params=pltpu.CompilerParams(needs_layout_passes=False),
    )
    return fn(cb_words, indices)


BM, BK = 2048, 1024
BKW = BK // 2
BN = ROWS_PER_CALL
_HI_MASK = -65536


def _mm_compute(x_ref, g_ref, s_ref, o_ref):
    k = pl.program_id(1)

    @pl.when(k == 0)
    def _zero():
        o_ref[...] = jnp.zeros_like(o_ref)

    w = g_ref[...]
    g_even = pltpu.bitcast(w << 16, jnp.float32).astype(jnp.bfloat16)
    g_odd = pltpu.bitcast(w & _HI_MASK, jnp.float32).astype(jnp.bfloat16)
    g2 = jnp.concatenate([g_even, g_odd], axis=1)
    o_ref[...] += lax.dot_general(
        x_ref[...], g2, (((1,), (1,)), ((), ())),
        preferred_element_type=jnp.float32,
        precision=lax.Precision.DEFAULT,
    )

    @pl.when(k == pl.num_programs(1) - 1)
    def _scale():
        o_ref[...] *= s_ref[...]


def _mm_body_first(x_ref, g_ref, s_ref, o_ref):
    _mm_compute(x_ref, g_ref, s_ref, o_ref)


def _mm_body_rest(x_ref, g_ref, s_ref, prev_ref, o_ref):
    _mm_compute(x_ref, g_ref, s_ref, o_ref)


def _tc_matmul_part(x, g_words, s_row, prev, col):
    m, kdim = x.shape
    grid = (m // BM, kdim // BK)
    in_specs = [
        pl.BlockSpec((BM, BK), lambda i, k: (i, k)),
        pl.BlockSpec((BN, BKW), lambda i, k: (0, k)),
        pl.BlockSpec((1, BN), lambda i, k: (0, 0)),
    ]
    args = (x, g_words, s_row)
    if prev is None:
        body, aliases = _mm_body_first, {}
    else:
        body, aliases = _mm_body_rest, {3: 0}
        in_specs.append(pl.BlockSpec(memory_space=pl.ANY))
        args = args + (prev,)
    return pl.pallas_call(
        body,
        grid=grid,
        in_specs=in_specs,
        out_specs=pl.BlockSpec((BM, BN), lambda i, k, c=col: (i, c)),
        out_shape=jax.ShapeDtypeStruct((m, WEIGHT_ROWS), jnp.float32),
        input_output_aliases=aliases,
        compiler_params=pltpu.CompilerParams(
            dimension_semantics=("parallel", "arbitrary"),
            vmem_limit_bytes=100 * 1024 * 1024,
        ),
    )(*args)


def kernel(x, indices, codebook, scales):
    cb_words = lax.bitcast_convert_type(
        codebook.astype(jnp.bfloat16).reshape(NUM_CODES, CODE_W, 2), jnp.int32
    ).reshape(-1)
    perm = (
        jnp.arange(WEIGHT_COLS, dtype=jnp.int32)
        .reshape(WEIGHT_COLS // BK, BKW, 2)
        .transpose(0, 2, 1)
        .reshape(-1)
    )
    x_bf = jnp.take(x, perm, axis=1).astype(jnp.bfloat16)
    s_row = scales.reshape(1, WEIGHT_ROWS)
    out = None
    for part in range(N_SPLIT):
        gw = _sc_gather(cb_words, indices, part * VECS_PER_CALL)
        s_part = lax.slice_in_dim(s_row, part * BN, (part + 1) * BN, axis=1)
        out = _tc_matmul_part(x_bf, gw, s_part, out, part)
    return out
